# baseline (speedup 1.0000x reference)
import functools

import jax
import jax.numpy as jnp
from jax import lax
from jax.experimental import pallas as pl
from jax.experimental.pallas import tpu as pltpu
from jax.experimental.pallas import tpu_sc as plsc

NC = 2
NS = 16
L = 16
NW = NC * NS

CH = 128


def _make_gather_kernel(B, K):
    assert B % NW == 0
    bw = B // NW
    nch = bw // CH
    assert nch * CH == bw and K % L == 0
    K2 = 2 * K

    mesh = plsc.VectorSubcoreMesh(core_axis_name="c", subcore_axis_name="s")

    @functools.partial(
        pl.kernel,
        mesh=mesh,
        out_type=jax.ShapeDtypeStruct((B,), jnp.float32),
        compiler_params=pltpu.CompilerParams(
            needs_layout_passes=False, use_tc_tiling_on_sc=False),
        scratch_types=[
            pltpu.VMEM((bw,), jnp.int32),
            pltpu.VMEM((bw,), jnp.int32),
            pltpu.VMEM((bw,), jnp.int32),
            pltpu.VMEM((bw,), jnp.int32),
            pltpu.VMEM((CH, K2), jnp.float32),
            pltpu.VMEM((CH, K2), jnp.float32),
            pltpu.VMEM((bw,), jnp.float32),
            pltpu.SemaphoreType.DMA,
        ],
    )
    def body(users_hbm, movies_hbm, upairs_hbm, mpairs_hbm, out_hbm,
             ug, mg, ub, mb, ubuf, mbuf, outv, sem):
        wid = lax.axis_index("s") * NC + lax.axis_index("c")
        base = wid * bw
        pltpu.sync_copy(users_hbm.at[pl.ds(base, bw)], ug)
        pltpu.sync_copy(movies_hbm.at[pl.ds(base, bw)], mg)
        lane = lax.iota(jnp.int32, L)

        def split(j, _):
            sl = pl.ds(j * L, L)
            uv, mv = ug[sl], mg[sl]
            ub[sl] = (uv & 1) * K
            mb[sl] = (mv & 1) * K
            ug[sl] = lax.shift_right_logical(uv, 1)
            mg[sl] = lax.shift_right_logical(mv, 1)
            return _

        lax.fori_loop(0, bw // L, split, 0)

        for c in range(nch):
            pltpu.async_copy(
                upairs_hbm.at[ug.at[pl.ds(c * CH, CH)]], ubuf, sem).wait()
            pltpu.async_copy(
                mpairs_hbm.at[mg.at[pl.ds(c * CH, CH)]], mbuf, sem).wait()

            def group(g, _, c=c):
                sl = pl.ds(c * CH + g * L, L)
                ubv, mbv = ub[sl], mb[sl]
                accv = jnp.zeros((L,), jnp.float32)
                for i in range(L):
                    r = g * L + i
                    ubase, mbase = ubv[i], mbv[i]
                    p = (ubuf[r, pl.ds(ubase, L)] * mbuf[r, pl.ds(mbase, L)])
                    for k in range(L, K, L):
                        p += (ubuf[r, pl.ds(ubase + k, L)]
                              * mbuf[r, pl.ds(mbase + k, L)])
                    accv = jnp.where(lane == i, plsc.cumsum(p)[L - 1], accv)
                outv[pl.ds(c * CH + g * L, L)] = accv
                return _

            lax.fori_loop(0, CH // L, group, 0)

        pltpu.sync_copy(outv, out_hbm.at[pl.ds(base, bw)])

    return body


def kernel(users, movies, user_emb, movie_emb):
    B = users.shape[0]
    K = user_emb.shape[1]
    ulin = lax.optimization_barrier(user_emb.reshape(-1))
    mlin = lax.optimization_barrier(movie_emb.reshape(-1))
    upairs = ulin.reshape(user_emb.shape[0] // 2, 2 * K)
    mpairs = mlin.reshape(movie_emb.shape[0] // 2, 2 * K)
    return _make_gather_kernel(B, K)(
        users.astype(jnp.int32), movies.astype(jnp.int32), upairs, mpairs)


# speedup vs baseline: 1.6547x; 1.6547x over previous
import functools

import jax
import jax.numpy as jnp
from jax import lax
from jax.experimental import pallas as pl
from jax.experimental.pallas import tpu as pltpu
from jax.experimental.pallas import tpu_sc as plsc

NC = 2
NS = 16
L = 16
NW = NC * NS


def _make_sc_kernel(B, K):
    assert B % NW == 0
    bw = B // NW
    ng = bw // L
    assert ng * L == bw and K % L == 0

    mesh = plsc.VectorSubcoreMesh(core_axis_name="c", subcore_axis_name="s")

    @functools.partial(
        pl.kernel,
        mesh=mesh,
        out_type=jax.ShapeDtypeStruct((B,), jnp.float32),
        compiler_params=pltpu.CompilerParams(
            needs_layout_passes=False, use_tc_tiling_on_sc=True),
        scratch_types=[
            pltpu.VMEM((bw,), jnp.int32),
            pltpu.VMEM((bw,), jnp.int32),
            pltpu.VMEM((2, L, K), jnp.float32),
            pltpu.VMEM((2, L, K), jnp.float32),
            pltpu.VMEM((bw,), jnp.float32),
            pltpu.SemaphoreType.DMA((2,)),
        ],
    )
    def body(users_hbm, movies_hbm, uemb_hbm, memb_hbm, out_hbm,
             uidx, midx, urows, mrows, outv, sem):
        wid = lax.axis_index("s") * NC + lax.axis_index("c")
        base = wid * bw
        pltpu.sync_copy(users_hbm.at[pl.ds(base, bw)], uidx)
        pltpu.sync_copy(movies_hbm.at[pl.ds(base, bw)], midx)
        lane = lax.iota(jnp.int32, L)

        def fetch(g, s):
            uvec = uidx[pl.ds(g * L, L)]
            mvec = midx[pl.ds(g * L, L)]
            for i in range(L):
                pltpu.async_copy(
                    uemb_hbm.at[uvec[i]], urows.at[s, i], sem.at[s])
                pltpu.async_copy(
                    memb_hbm.at[mvec[i]], mrows.at[s, i], sem.at[s])

        def drain(s):
            for _ in range(2 * L):
                pltpu.make_async_copy(
                    uemb_hbm.at[0], urows.at[0, 0], sem.at[s]).wait()

        fetch(0, 0)

        def group(g, carry):
            s = g % 2

            @pl.when(g + 1 < ng)
            def _():
                fetch(g + 1, (g + 1) % 2)

            drain(s)
            accv = jnp.zeros((L,), jnp.float32)
            for i in range(L):
                p = urows[s, i, pl.ds(0, L)] * mrows[s, i, pl.ds(0, L)]
                for k in range(L, K, L):
                    p += urows[s, i, pl.ds(k, L)] * mrows[s, i, pl.ds(k, L)]
                accv = jnp.where(lane == i, plsc.cumsum(p)[L - 1], accv)
            outv[pl.ds(g * L, L)] = accv
            return carry

        lax.fori_loop(0, ng, group, 0)
        pltpu.sync_copy(outv, out_hbm.at[pl.ds(base, bw)])

    return body


def kernel(users, movies, user_emb, movie_emb):
    B = users.shape[0]
    K = user_emb.shape[1]
    return _make_sc_kernel(B, K)(
        users.astype(jnp.int32), movies.astype(jnp.int32),
        user_emb, movie_emb)
